# baseline (speedup 1.0000x reference)
import functools

import jax
import jax.numpy as jnp
from jax import lax
from jax.experimental import pallas as pl
from jax.experimental.pallas import tpu as pltpu
from jax.experimental.pallas import tpu_sc as plsc

N = 10000
E = 320000
D = 128
H = 4
F = 32
HF = H * F

NC = 2
NS = 16
NW = NC * NS
K = 128
BLK = 8
CHUNKS = 80
EPAD = NW * K * CHUNKS
NPAD = 10240
ROWS_PER_TILE = NPAD // NS

PBLK = 2048
FBLK = 2000


def _proj_body(feat_ref, w_ref, alr_ref, fp_ref, lr_ref):
    fp = lax.dot_general(feat_ref[:], w_ref[:], (((1,), (1,)), ((), ())),
                         preferred_element_type=jnp.float32)
    fp_ref[:] = fp
    lr_ref[:] = jnp.dot(fp, alr_ref[:], preferred_element_type=jnp.float32)


def _proj_call(featp, w, alr):
    return pl.pallas_call(
        _proj_body,
        grid=(NPAD // PBLK,),
        in_specs=[
            pl.BlockSpec((PBLK, D), lambda i: (i, 0)),
            pl.BlockSpec((HF, D), lambda i: (0, 0)),
            pl.BlockSpec((HF, 16), lambda i: (0, 0)),
        ],
        out_specs=[
            pl.BlockSpec((PBLK, HF), lambda i: (i, 0)),
            pl.BlockSpec((PBLK, 16), lambda i: (i, 0)),
        ],
        out_shape=[
            jax.ShapeDtypeStruct((NPAD, HF), jnp.float32),
            jax.ShapeDtypeStruct((NPAD, 16), jnp.float32),
        ],
    )(featp, w, alr)


@functools.lru_cache(maxsize=1)
def _get_edge_kernel():
    mesh = plsc.VectorSubcoreMesh(core_axis_name="c", subcore_axis_name="s",
                                  num_cores=NC, num_subcores=NS)
    return pl.kernel(
        _edge_body,
        out_type=(
            jax.ShapeDtypeStruct((NC, NPAD, HF), jnp.float32),
            jax.ShapeDtypeStruct((NC, NPAD, 16), jnp.float32),
        ),
        mesh=mesh,
        compiler_params=pltpu.CompilerParams(use_tc_tiling_on_sc=False,
                                             needs_layout_passes=False),
        scratch_types=[
            pltpu.VMEM((BLK * K,), jnp.int32),
            pltpu.VMEM((BLK * K,), jnp.int32),
            pltpu.VMEM((BLK * K,), jnp.float32),
            pltpu.VMEM((K, 16), jnp.float32),
            pltpu.VMEM((K, 16), jnp.float32),
            pltpu.VMEM((K, HF), jnp.float32),
            pltpu.VMEM((K, 16), jnp.float32),
            pltpu.VMEM((H, K), jnp.float32),
            pltpu.VMEM((16,), jnp.float32),
            pltpu.VMEM((16,), jnp.float32),
            pltpu.VMEM_SHARED((NPAD, HF), jnp.float32),
            pltpu.VMEM_SHARED((NPAD, 16), jnp.float32),
            pltpu.VMEM_SHARED((NPAD, 16), jnp.float32),
            pltpu.SemaphoreType.DMA,
        ],
    )


def _lane_splat(v, lane):
    idx = jnp.full((16, 1), lane, jnp.int32)
    return lax.gather(v, idx,
                      lax.GatherDimensionNumbers((), (0,), (0,)), (1,),
                      mode=lax.GatherScatterMode.PROMISE_IN_BOUNDS)


def _edge_body(fp_hbm, lr_hbm, src_hbm, dst_hbm, ew_hbm, wv_hbm, bv_hbm,
               rst_out, esum_out,
               src_blk, dst_blk, ew_blk, lrs_v, lrd_v, fp_v, ee_v, eet_v,
               wv_v, bv_v, rst_sh, esum_sh, lr_sh, sem):
    cid = lax.axis_index("c")
    sid = lax.axis_index("s")
    wid = sid * NC + cid

    zeros16 = jnp.zeros((16,), jnp.float32)
    iota16 = lax.iota(jnp.int32, 16)

    pltpu.sync_copy(wv_hbm, wv_v)
    pltpu.sync_copy(bv_hbm, bv_v)
    wvec = wv_v[:]
    bvec = bv_v[:]

    def _zrow(r, _):
        for j in range(HF // 16):
            fp_v[r, pl.ds(j * 16, 16)] = zeros16
        return 0
    lax.fori_loop(0, K, _zrow, 0)

    for g in range(K // 16):
        rows = iota16 + g * 16
        for h in range(16):
            plsc.store_scatter(ee_v, [rows, jnp.full((16,), h, jnp.int32)], zeros16)

    base_r = sid * ROWS_PER_TILE
    for b in range(ROWS_PER_TILE // K):
        pltpu.sync_copy(fp_v, rst_sh.at[pl.ds(base_r + b * K, K)])
        pltpu.sync_copy(ee_v, esum_sh.at[pl.ds(base_r + b * K, K)])
        pltpu.sync_copy(lr_hbm.at[pl.ds(base_r + b * K, K)],
                        lr_sh.at[pl.ds(base_r + b * K, K)])
    plsc.subcore_barrier()

    def _chunk(ci, _):
        src_v = src_blk.at[pl.ds(ci * K, K)]
        dst_v = dst_blk.at[pl.ds(ci * K, K)]
        fp_cp = pltpu.async_copy(fp_hbm.at[src_v], fp_v, sem)
        pltpu.sync_copy(lr_sh.at[src_v], lrs_v)
        pltpu.sync_copy(lr_sh.at[dst_v], lrd_v)

        for g in range(K // 16):
            rows = iota16 + g * 16
            wfull = ew_blk[pl.ds(ci * K + g * 16, 16)] * wvec + bvec
            for h in range(H):
                els = plsc.load_gather(lrs_v, [rows, jnp.full((16,), h, jnp.int32)])
                erd = plsc.load_gather(lrd_v, [rows, jnp.full((16,), 4 + h, jnp.int32)])
                e = els + erd
                e = jnp.where(e > 0.0, e, 0.2 * e)
                ee = jnp.exp(e * wfull)
                plsc.store_scatter(ee_v, [rows, jnp.full((16,), h, jnp.int32)], ee)
                eet_v[h, pl.ds(g * 16, 16)] = ee

        pltpu.sync_copy(ee_v, esum_sh.at[dst_v], add=True)
        fp_cp.wait()

        def _scale(gb, _):
            cb = gb * 16
            eh = [eet_v[h, pl.ds(cb, 16)] for h in range(H)]
            for cl in range(16):
                for h in range(H):
                    s = _lane_splat(eh[h], cl)
                    for j in range(F // 16):
                        sl = pl.ds(h * F + j * 16, 16)
                        fp_v[cb + cl, sl] = fp_v[cb + cl, sl] * s
            return 0
        lax.fori_loop(0, K // 16, _scale, 0)

        pltpu.sync_copy(fp_v, rst_sh.at[dst_v], add=True)
        return 0

    def _block(bi, _):
        eb = (wid * CHUNKS + bi * BLK) * K
        pltpu.sync_copy(src_hbm.at[pl.ds(eb, BLK * K)], src_blk)
        pltpu.sync_copy(dst_hbm.at[pl.ds(eb, BLK * K)], dst_blk)
        pltpu.sync_copy(ew_hbm.at[pl.ds(eb, BLK * K)], ew_blk)
        lax.fori_loop(0, BLK, _chunk, 0)
        return 0

    lax.fori_loop(0, CHUNKS // BLK, _block, 0)

    plsc.subcore_barrier()
    for b in range(ROWS_PER_TILE // K):
        rs = base_r + b * K
        pltpu.sync_copy(rst_sh.at[pl.ds(rs, K)], rst_out.at[cid, pl.ds(rs, K)])
        pltpu.sync_copy(esum_sh.at[pl.ds(rs, K)], esum_out.at[cid, pl.ds(rs, K)])


def _final_body(r0_ref, r1_ref, e0_ref, e1_ref, s_ref, b_ref, o_ref):
    es = jnp.dot(e0_ref[:] + e1_ref[:], s_ref[:], preferred_element_type=jnp.float32)
    den = jnp.where(es > 0.0, es, 1.0)
    o_ref[:] = (r0_ref[:] + r1_ref[:]) / den + b_ref[:]


def _final_call(r0, r1, e0, e1, sel, brow):
    return pl.pallas_call(
        _final_body,
        grid=(N // FBLK,),
        in_specs=[
            pl.BlockSpec((FBLK, HF), lambda i: (i, 0)),
            pl.BlockSpec((FBLK, HF), lambda i: (i, 0)),
            pl.BlockSpec((FBLK, 16), lambda i: (i, 0)),
            pl.BlockSpec((FBLK, 16), lambda i: (i, 0)),
            pl.BlockSpec((16, HF), lambda i: (0, 0)),
            pl.BlockSpec((1, HF), lambda i: (0, 0)),
        ],
        out_specs=pl.BlockSpec((FBLK, HF), lambda i: (i, 0)),
        out_shape=jax.ShapeDtypeStruct((N, HF), jnp.float32),
    )(r0, r1, e0, e1, sel, brow)


def kernel(feat, edge_index, edge_weight, W, attn_l, attn_r, w_lin_w, w_lin_b, bias):
    featp = jnp.pad(feat, ((0, NPAD - N), (0, 0)))
    src = edge_index[0]
    dst = edge_index[1]
    srcp = jnp.pad(src, (0, EPAD - E))
    dstp = jnp.pad(dst, (0, EPAD - E), constant_values=N)
    ewp = jnp.pad(edge_weight, (0, EPAD - E))

    r = jnp.arange(HF)
    alr = (jnp.zeros((HF, 16), jnp.float32)
           .at[r, r // F].set(attn_l.reshape(HF))
           .at[r, 4 + r // F].set(attn_r.reshape(HF)))
    wv = jnp.full((16,), w_lin_w[0, 0], jnp.float32)
    bv = jnp.full((16,), w_lin_b[0], jnp.float32)
    sel = (jnp.arange(HF)[None, :] // F == jnp.arange(16)[:, None]).astype(jnp.float32)

    fp, lr = _proj_call(featp, W, alr)
    rst2, esum2 = _get_edge_kernel()(fp, lr, srcp, dstp, ewp, wv, bv)
    out = _final_call(rst2[0], rst2[1], esum2[0], esum2[1], sel, bias.reshape(1, HF))
    return out.reshape(N, H, F)


# speedup vs baseline: 1.1331x; 1.1331x over previous
import functools

import jax
import jax.numpy as jnp
from jax import lax
from jax.experimental import pallas as pl
from jax.experimental.pallas import tpu as pltpu
from jax.experimental.pallas import tpu_sc as plsc

N = 10000
E = 320000
D = 128
H = 4
F = 32
HF = H * F

NC = 2
NS = 16
NW = NC * NS
K = 128
CHUNKS = (E + NW * K - 1) // (NW * K)
EPAD = NW * K * CHUNKS
NPAD = 10240
ROWS_PER_TILE = NPAD // NS

PBLK = 2048
FBLK = 2000


def _proj_body(feat_ref, w_ref, alr_ref, fp_ref, lr_ref):
    fp = lax.dot_general(feat_ref[:], w_ref[:], (((1,), (1,)), ((), ())),
                         preferred_element_type=jnp.float32)
    fp_ref[:] = fp
    lr_ref[:] = jnp.dot(fp, alr_ref[:], preferred_element_type=jnp.float32)


def _proj_call(featp, w, alr):
    return pl.pallas_call(
        _proj_body,
        grid=(NPAD // PBLK,),
        in_specs=[
            pl.BlockSpec((PBLK, D), lambda i: (i, 0)),
            pl.BlockSpec((HF, D), lambda i: (0, 0)),
            pl.BlockSpec((HF, 16), lambda i: (0, 0)),
        ],
        out_specs=[
            pl.BlockSpec((PBLK, HF), lambda i: (i, 0)),
            pl.BlockSpec((PBLK, 16), lambda i: (i, 0)),
        ],
        out_shape=[
            jax.ShapeDtypeStruct((NPAD, HF), jnp.float32),
            jax.ShapeDtypeStruct((NPAD, 16), jnp.float32),
        ],
    )(featp, w, alr)


@functools.lru_cache(maxsize=1)
def _get_edge_kernel():
    mesh = plsc.VectorSubcoreMesh(core_axis_name="c", subcore_axis_name="s",
                                  num_cores=NC, num_subcores=NS)
    return pl.kernel(
        _edge_body,
        out_type=(
            jax.ShapeDtypeStruct((NC, NPAD, HF), jnp.float32),
            jax.ShapeDtypeStruct((NC, NPAD, 16), jnp.float32),
        ),
        mesh=mesh,
        compiler_params=pltpu.CompilerParams(use_tc_tiling_on_sc=False,
                                             needs_layout_passes=False),
        scratch_types=[
            pltpu.VMEM((K,), jnp.int32),
            pltpu.VMEM((K,), jnp.int32),
            pltpu.VMEM((K,), jnp.float32),
            pltpu.VMEM((K, 16), jnp.float32),
            pltpu.VMEM((K, 16), jnp.float32),
            pltpu.VMEM((K, HF), jnp.float32),
            pltpu.VMEM((K, 16), jnp.float32),
            pltpu.VMEM((H, K), jnp.float32),
            pltpu.VMEM((16,), jnp.float32),
            pltpu.VMEM((16,), jnp.float32),
            pltpu.VMEM_SHARED((NPAD, HF), jnp.float32),
            pltpu.VMEM_SHARED((NPAD, 16), jnp.float32),
            pltpu.VMEM_SHARED((NPAD, 16), jnp.float32),
            pltpu.SemaphoreType.DMA,
        ],
    )


def _lane_splat(v, lane):
    idx = jnp.full((16, 1), lane, jnp.int32)
    return lax.gather(v, idx,
                      lax.GatherDimensionNumbers((), (0,), (0,)), (1,),
                      mode=lax.GatherScatterMode.PROMISE_IN_BOUNDS)


def _edge_body(fp_hbm, lr_hbm, src_hbm, dst_hbm, ew_hbm, wv_hbm, bv_hbm,
               rst_out, esum_out,
               src_v, dst_v, ew_v, lrs_v, lrd_v, fp_v, ee_v, eet_v,
               wv_v, bv_v, rst_sh, esum_sh, lr_sh, sem):
    cid = lax.axis_index("c")
    sid = lax.axis_index("s")
    wid = sid * NC + cid

    zeros16 = jnp.zeros((16,), jnp.float32)
    iota16 = lax.iota(jnp.int32, 16)

    pltpu.sync_copy(wv_hbm, wv_v)
    pltpu.sync_copy(bv_hbm, bv_v)
    wvec = wv_v[:]
    bvec = bv_v[:]

    def _zrow(r, _):
        for j in range(HF // 16):
            fp_v[r, pl.ds(j * 16, 16)] = zeros16
        return 0
    lax.fori_loop(0, K, _zrow, 0)

    for g in range(K // 16):
        rows = iota16 + g * 16
        for h in range(16):
            plsc.store_scatter(ee_v, [rows, jnp.full((16,), h, jnp.int32)], zeros16)

    base_r = sid * ROWS_PER_TILE
    for b in range(ROWS_PER_TILE // K):
        pltpu.sync_copy(fp_v, rst_sh.at[pl.ds(base_r + b * K, K)])
        pltpu.sync_copy(ee_v, esum_sh.at[pl.ds(base_r + b * K, K)])
        pltpu.sync_copy(lr_hbm.at[pl.ds(base_r + b * K, K)],
                        lr_sh.at[pl.ds(base_r + b * K, K)])
    plsc.subcore_barrier()

    def _chunk(ci, _):
        base = (wid * CHUNKS + ci) * K
        pltpu.sync_copy(src_hbm.at[pl.ds(base, K)], src_v)
        pltpu.sync_copy(dst_hbm.at[pl.ds(base, K)], dst_v)
        pltpu.sync_copy(ew_hbm.at[pl.ds(base, K)], ew_v)
        fp_cp = pltpu.async_copy(fp_hbm.at[src_v], fp_v, sem)
        pltpu.sync_copy(lr_sh.at[src_v], lrs_v)
        pltpu.sync_copy(lr_sh.at[dst_v], lrd_v)

        for g in range(K // 16):
            rows = iota16 + g * 16
            wfull = ew_v[pl.ds(g * 16, 16)] * wvec + bvec
            for h in range(H):
                els = plsc.load_gather(lrs_v, [rows, jnp.full((16,), h, jnp.int32)])
                erd = plsc.load_gather(lrd_v, [rows, jnp.full((16,), 4 + h, jnp.int32)])
                e = els + erd
                e = jnp.where(e > 0.0, e, 0.2 * e)
                ee = jnp.exp(e * wfull)
                plsc.store_scatter(ee_v, [rows, jnp.full((16,), h, jnp.int32)], ee)
                eet_v[h, pl.ds(g * 16, 16)] = ee

        pltpu.sync_copy(ee_v, esum_sh.at[dst_v], add=True)
        fp_cp.wait()

        def _scale(gb, _):
            cb = gb * 16
            eh = [eet_v[h, pl.ds(cb, 16)] for h in range(H)]
            for cl in range(16):
                for h in range(H):
                    s = _lane_splat(eh[h], cl)
                    for j in range(F // 16):
                        sl = pl.ds(h * F + j * 16, 16)
                        fp_v[cb + cl, sl] = fp_v[cb + cl, sl] * s
            return 0
        lax.fori_loop(0, K // 16, _scale, 0)

        pltpu.sync_copy(fp_v, rst_sh.at[dst_v], add=True)
        return 0

    lax.fori_loop(0, CHUNKS, _chunk, 0)

    plsc.subcore_barrier()
    for b in range(ROWS_PER_TILE // K):
        rs = base_r + b * K
        pltpu.sync_copy(rst_sh.at[pl.ds(rs, K)], rst_out.at[cid, pl.ds(rs, K)])
        pltpu.sync_copy(esum_sh.at[pl.ds(rs, K)], esum_out.at[cid, pl.ds(rs, K)])


def _final_body(r0_ref, r1_ref, e0_ref, e1_ref, s_ref, b_ref, o_ref):
    es = jnp.dot(e0_ref[:] + e1_ref[:], s_ref[:], preferred_element_type=jnp.float32)
    den = jnp.where(es > 0.0, es, 1.0)
    o_ref[:] = (r0_ref[:] + r1_ref[:]) / den + b_ref[:]


def _final_call(r0, r1, e0, e1, sel, brow):
    return pl.pallas_call(
        _final_body,
        grid=(N // FBLK,),
        in_specs=[
            pl.BlockSpec((FBLK, HF), lambda i: (i, 0)),
            pl.BlockSpec((FBLK, HF), lambda i: (i, 0)),
            pl.BlockSpec((FBLK, 16), lambda i: (i, 0)),
            pl.BlockSpec((FBLK, 16), lambda i: (i, 0)),
            pl.BlockSpec((16, HF), lambda i: (0, 0)),
            pl.BlockSpec((1, HF), lambda i: (0, 0)),
        ],
        out_specs=pl.BlockSpec((FBLK, HF), lambda i: (i, 0)),
        out_shape=jax.ShapeDtypeStruct((N, HF), jnp.float32),
    )(r0, r1, e0, e1, sel, brow)


def kernel(feat, edge_index, edge_weight, W, attn_l, attn_r, w_lin_w, w_lin_b, bias):
    featp = jnp.pad(feat, ((0, NPAD - N), (0, 0)))
    src = edge_index[0]
    dst = edge_index[1]
    srcp = jnp.pad(src, (0, EPAD - E))
    dstp = jnp.concatenate(
        [dst, N + jnp.arange(EPAD - E, dtype=jnp.int32) % (NPAD - N)])
    ewp = jnp.pad(edge_weight, (0, EPAD - E))

    r = jnp.arange(HF)
    alr = (jnp.zeros((HF, 16), jnp.float32)
           .at[r, r // F].set(attn_l.reshape(HF))
           .at[r, 4 + r // F].set(attn_r.reshape(HF)))
    wv = jnp.full((16,), w_lin_w[0, 0], jnp.float32)
    bv = jnp.full((16,), w_lin_b[0], jnp.float32)
    sel = (jnp.arange(HF)[None, :] // F == jnp.arange(16)[:, None]).astype(jnp.float32)

    fp, lr = _proj_call(featp, W, alr)
    rst2, esum2 = _get_edge_kernel()(fp, lr, srcp, dstp, ewp, wv, bv)
    out = _final_call(rst2[0], rst2[1], esum2[0], esum2[1], sel, bias.reshape(1, HF))
    return out.reshape(N, H, F)


# speedup vs baseline: 1.2817x; 1.1312x over previous
import functools

import jax
import jax.numpy as jnp
from jax import lax
from jax.experimental import pallas as pl
from jax.experimental.pallas import tpu as pltpu
from jax.experimental.pallas import tpu_sc as plsc

N = 10000
E = 320000
D = 128
H = 4
F = 32
HF = H * F

NC = 2
NS = 16
NW = NC * NS
K = 128
CHUNKS = (E + NW * K - 1) // (NW * K)
EPAD = NW * K * CHUNKS
NPAD = 10240
ROWS_PER_TILE = NPAD // NS

PBLK = 2048
FBLK = 2000


def _proj_body(feat_ref, w_ref, alr_ref, fp_ref, lr_ref):
    fp = lax.dot_general(feat_ref[:], w_ref[:], (((1,), (1,)), ((), ())),
                         preferred_element_type=jnp.float32)
    fp_ref[:] = fp
    lr_ref[:] = jnp.dot(fp, alr_ref[:], preferred_element_type=jnp.float32)


def _proj_call(featp, w, alr):
    return pl.pallas_call(
        _proj_body,
        grid=(NPAD // PBLK,),
        in_specs=[
            pl.BlockSpec((PBLK, D), lambda i: (i, 0)),
            pl.BlockSpec((HF, D), lambda i: (0, 0)),
            pl.BlockSpec((HF, 16), lambda i: (0, 0)),
        ],
        out_specs=[
            pl.BlockSpec((PBLK, HF), lambda i: (i, 0)),
            pl.BlockSpec((PBLK, 16), lambda i: (i, 0)),
        ],
        out_shape=[
            jax.ShapeDtypeStruct((NPAD, HF), jnp.float32),
            jax.ShapeDtypeStruct((NPAD, 16), jnp.float32),
        ],
    )(featp, w, alr)


@functools.lru_cache(maxsize=1)
def _get_edge_kernel():
    mesh = plsc.VectorSubcoreMesh(core_axis_name="c", subcore_axis_name="s",
                                  num_cores=NC, num_subcores=NS)
    return pl.kernel(
        _edge_body,
        out_type=(
            jax.ShapeDtypeStruct((NC, NPAD, HF), jnp.float32),
            jax.ShapeDtypeStruct((NC, NPAD, 16), jnp.float32),
        ),
        mesh=mesh,
        compiler_params=pltpu.CompilerParams(use_tc_tiling_on_sc=False,
                                             needs_layout_passes=False),
        scratch_types=[
            pltpu.VMEM((K,), jnp.int32),
            pltpu.VMEM((K,), jnp.int32),
            pltpu.VMEM((K,), jnp.float32),
            pltpu.VMEM((K, 16), jnp.float32),
            pltpu.VMEM((K, 16), jnp.float32),
            pltpu.VMEM((K, HF), jnp.float32),
            pltpu.VMEM((K, 16), jnp.float32),
            pltpu.VMEM((H, K), jnp.float32),
            pltpu.VMEM((16,), jnp.float32),
            pltpu.VMEM((16,), jnp.float32),
            pltpu.VMEM_SHARED((NPAD, HF), jnp.float32),
            pltpu.VMEM_SHARED((NPAD, 16), jnp.float32),
            pltpu.VMEM_SHARED((NPAD, 16), jnp.float32),
            pltpu.SemaphoreType.DMA,
            pltpu.SemaphoreType.DMA,
            pltpu.SemaphoreType.DMA,
            pltpu.SemaphoreType.DMA,
        ],
    )


def _lane_splat(v, lane):
    idx = jnp.full((16, 1), lane, jnp.int32)
    return lax.gather(v, idx,
                      lax.GatherDimensionNumbers((), (0,), (0,)), (1,),
                      mode=lax.GatherScatterMode.PROMISE_IN_BOUNDS)


def _edge_body(fp_hbm, lr_hbm, src_hbm, dst_hbm, ew_hbm, wv_hbm, bv_hbm,
               rst_out, esum_out,
               src_v, dst_v, ew_v, lrs_v, lrd_v, fp_v, ee_v, eet_v,
               wv_v, bv_v, rst_sh, esum_sh, lr_sh, sem, sem_s, sem_d, sem_w):
    cid = lax.axis_index("c")
    sid = lax.axis_index("s")
    wid = sid * NC + cid

    zeros16 = jnp.zeros((16,), jnp.float32)
    iota16 = lax.iota(jnp.int32, 16)

    pltpu.sync_copy(wv_hbm, wv_v)
    pltpu.sync_copy(bv_hbm, bv_v)
    wvec = wv_v[:]
    bvec = bv_v[:]

    def _zrow(r, _):
        for j in range(HF // 16):
            fp_v[r, pl.ds(j * 16, 16)] = zeros16
        return 0
    lax.fori_loop(0, K, _zrow, 0)

    for g in range(K // 16):
        rows = iota16 + g * 16
        for h in range(16):
            plsc.store_scatter(ee_v, [rows, jnp.full((16,), h, jnp.int32)], zeros16)

    base_r = sid * ROWS_PER_TILE
    for b in range(ROWS_PER_TILE // K):
        pltpu.sync_copy(fp_v, rst_sh.at[pl.ds(base_r + b * K, K)])
        pltpu.sync_copy(ee_v, esum_sh.at[pl.ds(base_r + b * K, K)])
        pltpu.sync_copy(lr_hbm.at[pl.ds(base_r + b * K, K)],
                        lr_sh.at[pl.ds(base_r + b * K, K)])
    plsc.subcore_barrier()

    def _chunk(ci, _):
        base = (wid * CHUNKS + ci) * K
        s_cp = pltpu.async_copy(src_hbm.at[pl.ds(base, K)], src_v, sem_s)
        d_cp = pltpu.async_copy(dst_hbm.at[pl.ds(base, K)], dst_v, sem_d)
        w_cp = pltpu.async_copy(ew_hbm.at[pl.ds(base, K)], ew_v, sem_w)
        s_cp.wait()
        fp_cp = pltpu.async_copy(fp_hbm.at[src_v], fp_v, sem)
        pltpu.sync_copy(lr_sh.at[src_v], lrs_v)
        d_cp.wait()
        pltpu.sync_copy(lr_sh.at[dst_v], lrd_v)
        w_cp.wait()

        for g in range(K // 16):
            rows = iota16 + g * 16
            wfull = ew_v[pl.ds(g * 16, 16)] * wvec + bvec
            for h in range(H):
                els = plsc.load_gather(lrs_v, [rows, jnp.full((16,), h, jnp.int32)])
                erd = plsc.load_gather(lrd_v, [rows, jnp.full((16,), 4 + h, jnp.int32)])
                e = els + erd
                e = jnp.where(e > 0.0, e, 0.2 * e)
                ee = jnp.exp(e * wfull)
                plsc.store_scatter(ee_v, [rows, jnp.full((16,), h, jnp.int32)], ee)
                eet_v[h, pl.ds(g * 16, 16)] = ee

        pltpu.sync_copy(ee_v, esum_sh.at[dst_v], add=True)
        fp_cp.wait()

        def _scale(gb, _):
            cb = gb * 16
            eh = [eet_v[h, pl.ds(cb, 16)] for h in range(H)]
            for cl in range(16):
                for h in range(H):
                    s = _lane_splat(eh[h], cl)
                    for j in range(F // 16):
                        sl = pl.ds(h * F + j * 16, 16)
                        fp_v[cb + cl, sl] = fp_v[cb + cl, sl] * s
            return 0
        lax.fori_loop(0, K // 16, _scale, 0)

        pltpu.sync_copy(fp_v, rst_sh.at[dst_v], add=True)
        return 0

    lax.fori_loop(0, CHUNKS, _chunk, 0)

    plsc.subcore_barrier()
    for b in range(ROWS_PER_TILE // K):
        rs = base_r + b * K
        pltpu.sync_copy(rst_sh.at[pl.ds(rs, K)], rst_out.at[cid, pl.ds(rs, K)])
        pltpu.sync_copy(esum_sh.at[pl.ds(rs, K)], esum_out.at[cid, pl.ds(rs, K)])


def _final_body(r0_ref, r1_ref, e0_ref, e1_ref, s_ref, b_ref, o_ref):
    es = jnp.dot(e0_ref[:] + e1_ref[:], s_ref[:], preferred_element_type=jnp.float32)
    den = jnp.where(es > 0.0, es, 1.0)
    o_ref[:] = (r0_ref[:] + r1_ref[:]) / den + b_ref[:]


def _final_call(r0, r1, e0, e1, sel, brow):
    return pl.pallas_call(
        _final_body,
        grid=(N // FBLK,),
        in_specs=[
            pl.BlockSpec((FBLK, HF), lambda i: (i, 0)),
            pl.BlockSpec((FBLK, HF), lambda i: (i, 0)),
            pl.BlockSpec((FBLK, 16), lambda i: (i, 0)),
            pl.BlockSpec((FBLK, 16), lambda i: (i, 0)),
            pl.BlockSpec((16, HF), lambda i: (0, 0)),
            pl.BlockSpec((1, HF), lambda i: (0, 0)),
        ],
        out_specs=pl.BlockSpec((FBLK, HF), lambda i: (i, 0)),
        out_shape=jax.ShapeDtypeStruct((N, HF), jnp.float32),
    )(r0, r1, e0, e1, sel, brow)


def kernel(feat, edge_index, edge_weight, W, attn_l, attn_r, w_lin_w, w_lin_b, bias):
    featp = jnp.pad(feat, ((0, NPAD - N), (0, 0)))
    src = edge_index[0]
    dst = edge_index[1]
    srcp = jnp.pad(src, (0, EPAD - E))
    dstp = jnp.concatenate(
        [dst, N + jnp.arange(EPAD - E, dtype=jnp.int32) % (NPAD - N)])
    ewp = jnp.pad(edge_weight, (0, EPAD - E))

    r = jnp.arange(HF)
    alr = (jnp.zeros((HF, 16), jnp.float32)
           .at[r, r // F].set(attn_l.reshape(HF))
           .at[r, 4 + r // F].set(attn_r.reshape(HF)))
    wv = jnp.full((16,), w_lin_w[0, 0], jnp.float32)
    bv = jnp.full((16,), w_lin_b[0], jnp.float32)
    sel = (jnp.arange(HF)[None, :] // F == jnp.arange(16)[:, None]).astype(jnp.float32)

    fp, lr = _proj_call(featp, W, alr)
    rst2, esum2 = _get_edge_kernel()(fp, lr, srcp, dstp, ewp, wv, bv)
    out = _final_call(rst2[0], rst2[1], esum2[0], esum2[1], sel, bias.reshape(1, HF))
    return out.reshape(N, H, F)
